# baseline (speedup 1.0000x reference)
import functools

import jax
import jax.numpy as jnp
from jax.experimental import pallas as pl
from jax.experimental.pallas import tpu as pltpu

N_EXPERTS = 8
TOP_K = 2
TOKEN_BLOCK = 8192


def _router_kernel(x_ref, w_ref, probs_ref, idx_ref, vals_ref):
    x = x_ref[...]
    w = w_ref[...]
    lg = jax.lax.dot_general(
        w, x, (((1,), (1,)), ((), ())), preferred_element_type=jnp.float32
    )
    m = jnp.max(lg, axis=0, keepdims=True)
    e = jnp.exp(lg - m)
    s = jnp.sum(e, axis=0, keepdims=True)
    p = e / s
    probs_ref[...] = p

    row = jax.lax.broadcasted_iota(jnp.int32, p.shape, 0)
    v1 = jnp.max(p, axis=0, keepdims=True)
    i1 = jnp.min(jnp.where(p == v1, row, N_EXPERTS), axis=0, keepdims=True)
    masked = jnp.where(row == i1, -jnp.inf, p)
    v2 = jnp.max(masked, axis=0, keepdims=True)
    i2 = jnp.min(jnp.where(masked == v2, row, N_EXPERTS), axis=0, keepdims=True)
    idx_ref[...] = jnp.concatenate([i1, i2], axis=0)
    vals_ref[...] = jnp.concatenate([v1, v2], axis=0)


@functools.partial(jax.jit, static_argnames=())
def kernel(x, w):
    n_tokens, d_model = x.shape
    grid = (n_tokens // TOKEN_BLOCK,)
    out_shapes = (
        jax.ShapeDtypeStruct((N_EXPERTS, n_tokens), jnp.float32),
        jax.ShapeDtypeStruct((TOP_K, n_tokens), jnp.int32),
        jax.ShapeDtypeStruct((TOP_K, n_tokens), jnp.float32),
    )
    probs_t, idx_t, vals_t = pl.pallas_call(
        _router_kernel,
        grid=grid,
        in_specs=[
            pl.BlockSpec((TOKEN_BLOCK, d_model), lambda i: (i, 0)),
            pl.BlockSpec((N_EXPERTS, d_model), lambda i: (0, 0)),
        ],
        out_specs=(
            pl.BlockSpec((N_EXPERTS, TOKEN_BLOCK), lambda i: (0, i)),
            pl.BlockSpec((TOP_K, TOKEN_BLOCK), lambda i: (0, i)),
            pl.BlockSpec((TOP_K, TOKEN_BLOCK), lambda i: (0, i)),
        ),
        out_shape=out_shapes,
        compiler_params=pltpu.CompilerParams(
            dimension_semantics=("parallel",),
        ),
    )(x, w)
    return (probs_t.T, idx_t.T, vals_t.T)


# speedup vs baseline: 1.0288x; 1.0288x over previous
import functools

import jax
import jax.numpy as jnp
from jax.experimental import pallas as pl
from jax.experimental.pallas import tpu as pltpu

N_EXPERTS = 8
TOP_K = 2
TOKEN_BLOCK = 8192


def _router_kernel(xa_ref, xb_ref, w_ref, probs_ref, idx_ref, vals_ref):
    w = w_ref[...]
    ha = xa_ref.shape[1]
    lg = jax.lax.dot_general(
        w[:, :ha], xa_ref[...], (((1,), (1,)), ((), ())),
        preferred_element_type=jnp.float32,
    ) + jax.lax.dot_general(
        w[:, ha:], xb_ref[...], (((1,), (1,)), ((), ())),
        preferred_element_type=jnp.float32,
    )
    m = jnp.max(lg, axis=0, keepdims=True)
    e = jnp.exp(lg - m)
    s = jnp.sum(e, axis=0, keepdims=True)
    p = e / s
    probs_ref[...] = p

    row = jax.lax.broadcasted_iota(jnp.int32, p.shape, 0)
    v1 = jnp.max(p, axis=0, keepdims=True)
    i1 = jnp.min(jnp.where(p == v1, row, N_EXPERTS), axis=0, keepdims=True)
    masked = jnp.where(row == i1, -jnp.inf, p)
    v2 = jnp.max(masked, axis=0, keepdims=True)
    i2 = jnp.min(jnp.where(masked == v2, row, N_EXPERTS), axis=0, keepdims=True)
    idx_ref[...] = jnp.concatenate([i1, i2], axis=0)
    vals_ref[...] = jnp.concatenate([v1, v2], axis=0)


@functools.partial(jax.jit, static_argnames=())
def kernel(x, w):
    n_tokens, d_model = x.shape
    grid = (n_tokens // TOKEN_BLOCK,)
    out_shapes = (
        jax.ShapeDtypeStruct((N_EXPERTS, n_tokens), jnp.float32),
        jax.ShapeDtypeStruct((TOP_K, n_tokens), jnp.int32),
        jax.ShapeDtypeStruct((TOP_K, n_tokens), jnp.float32),
    )
    probs_t, idx_t, vals_t = pl.pallas_call(
        _router_kernel,
        grid=grid,
        in_specs=[
            pl.BlockSpec((TOKEN_BLOCK, d_model // 2), lambda i: (i, 0)),
            pl.BlockSpec((TOKEN_BLOCK, d_model // 2), lambda i: (i, 1)),
            pl.BlockSpec((N_EXPERTS, d_model), lambda i: (0, 0)),
        ],
        out_specs=(
            pl.BlockSpec((N_EXPERTS, TOKEN_BLOCK), lambda i: (0, i)),
            pl.BlockSpec((TOP_K, TOKEN_BLOCK), lambda i: (0, i)),
            pl.BlockSpec((TOP_K, TOKEN_BLOCK), lambda i: (0, i)),
        ),
        out_shape=out_shapes,
        compiler_params=pltpu.CompilerParams(
            dimension_semantics=("parallel",),
        ),
    )(x, x, w)
    return (probs_t.T, idx_t.T, vals_t.T)


# speedup vs baseline: 1.1111x; 1.0800x over previous
import functools

import jax
import jax.numpy as jnp
from jax.experimental import pallas as pl
from jax.experimental.pallas import tpu as pltpu

N_EXPERTS = 8
TOP_K = 2
CHUNK = 1024
N_BUF = 4


def _chunk_compute(w, x_chunk, probs_ref, idx_ref, vals_ref, base):
    lg = jax.lax.dot_general(
        w, x_chunk, (((1,), (1,)), ((), ())),
        preferred_element_type=jnp.float32,
    )
    m = jnp.max(lg, axis=0, keepdims=True)
    e = jnp.exp(lg - m)
    s = jnp.sum(e, axis=0, keepdims=True)
    p = e / s
    probs_ref[:, pl.ds(base, CHUNK)] = p

    row = jax.lax.broadcasted_iota(jnp.int32, p.shape, 0)
    v1 = jnp.max(p, axis=0, keepdims=True)
    i1 = jnp.min(jnp.where(p == v1, row, N_EXPERTS), axis=0, keepdims=True)
    masked = jnp.where(row == i1, -jnp.inf, p)
    v2 = jnp.max(masked, axis=0, keepdims=True)
    i2 = jnp.min(jnp.where(masked == v2, row, N_EXPERTS), axis=0, keepdims=True)
    idx_ref[:, pl.ds(base, CHUNK)] = jnp.concatenate([i1, i2], axis=0)
    vals_ref[:, pl.ds(base, CHUNK)] = jnp.concatenate([v1, v2], axis=0)


def _router_kernel(x_hbm, w_ref, probs_ref, idx_ref, vals_ref, buf, sems):
    n_tokens = x_hbm.shape[0]
    n_chunks = n_tokens // CHUNK
    w = w_ref[...]

    def start_copy(c, slot):
        pltpu.make_async_copy(
            x_hbm.at[pl.ds(c * CHUNK, CHUNK), :],
            buf.at[slot],
            sems.at[slot],
        ).start()

    for c in range(min(N_BUF, n_chunks)):
        start_copy(c, c)
    for c in range(n_chunks):
        slot = c % N_BUF
        pltpu.make_async_copy(
            x_hbm.at[pl.ds(c * CHUNK, CHUNK), :],
            buf.at[slot],
            sems.at[slot],
        ).wait()
        _chunk_compute(w, buf[slot], probs_ref, idx_ref, vals_ref, c * CHUNK)
        nxt = c + N_BUF
        if nxt < n_chunks:
            start_copy(nxt, slot)


@functools.partial(jax.jit, static_argnames=())
def kernel(x, w):
    n_tokens, d_model = x.shape
    out_shapes = (
        jax.ShapeDtypeStruct((N_EXPERTS, n_tokens), jnp.float32),
        jax.ShapeDtypeStruct((TOP_K, n_tokens), jnp.int32),
        jax.ShapeDtypeStruct((TOP_K, n_tokens), jnp.float32),
    )
    probs_t, idx_t, vals_t = pl.pallas_call(
        _router_kernel,
        in_specs=[
            pl.BlockSpec(memory_space=pltpu.HBM),
            pl.BlockSpec(memory_space=pltpu.VMEM),
        ],
        out_specs=(
            pl.BlockSpec(memory_space=pltpu.VMEM),
            pl.BlockSpec(memory_space=pltpu.VMEM),
            pl.BlockSpec(memory_space=pltpu.VMEM),
        ),
        out_shape=out_shapes,
        scratch_shapes=[
            pltpu.VMEM((N_BUF, CHUNK, 768), jnp.float32),
            pltpu.SemaphoreType.DMA((N_BUF,)),
        ],
    )(x, w)
    return (probs_t.T, idx_t.T, vals_t.T)
